# baseline (speedup 1.0000x reference)
import jax
import jax.numpy as jnp
from jax import lax
from jax.experimental import pallas as pl
from jax.experimental.pallas import tpu as pltpu
from jax.experimental.pallas import tpu_sc as plsc

EMB_DIM = 100
PAD_DIM = 128
BATCH = 16384

NUM_CORES = 2
NUM_SUBCORES = 16
LANES = 16
NUM_WORKERS = NUM_CORES * NUM_SUBCORES

B_PER_W = BATCH // NUM_WORKERS
CHUNK = 128
CHUNKS_PER_HALF = B_PER_W // CHUNK
GROUPS = CHUNK // LANES
SEGS = PAD_DIM // LANES
NSTEPS = 2 * CHUNKS_PER_HALF


def _worker_base():
    return (lax.axis_index("c") * NUM_SUBCORES + lax.axis_index("s")) * B_PER_W


def _sc1_body(idx_hbm, ent_hbm, d_hbm,
              ih0, it0, ih1, it1, hb0, tb0, hb1, tb1, sem0, sem1):
    base = _worker_base()
    slots = (((ih0, it0), (hb0, tb0), sem0), ((ih1, it1), (hb1, tb1), sem1))

    def issue(step):
        ibufs, dbufs, sem = slots[step % 2]
        half, c = divmod(step, CHUNKS_PER_HALF)
        off = base + c * CHUNK
        cps = []
        for j, col in enumerate((3 * half, 3 * half + 2)):
            pltpu.sync_copy(idx_hbm.at[col, pl.ds(off, CHUNK)], ibufs[j])
            cp = pltpu.make_async_copy(ent_hbm.at[ibufs[j]], dbufs[j], sem)
            cp.start()
            cps.append(cp)
        return cps

    def compute_store(step):
        _, (hb, tb), _ = slots[step % 2]
        half, c = divmod(step, CHUNKS_PER_HALF)

        def row_body(b, carry):
            for j in range(SEGS):
                sl = pl.ds(LANES * j, LANES)
                hb[b, sl] = hb[b, sl] - tb[b, sl]
            return carry

        lax.fori_loop(0, CHUNK, row_body, 0)
        dst = half * BATCH + base + c * CHUNK
        pltpu.sync_copy(hb, d_hbm.at[pl.ds(dst, CHUNK), :])

    cps = issue(0)
    for step in range(NSTEPS):
        nxt = issue(step + 1) if step + 1 < NSTEPS else None
        for cp in cps:
            cp.wait()
        compute_store(step)
        cps = nxt


def _sc2_body(idx_hbm, rel_hbm, d_hbm, correct_hbm, corrupt_hbm,
              ir0, ir1, rb0, db0, rb1, db1, co_v, cu_v, sem0, sem1):
    base = _worker_base()
    slots = ((ir0, (rb0, db0), sem0), (ir1, (rb1, db1), sem1))

    def issue(step):
        ibuf, (rb, db), sem = slots[step % 2]
        half, c = divmod(step, CHUNKS_PER_HALF)
        off = base + c * CHUNK
        pltpu.sync_copy(idx_hbm.at[3 * half + 1, pl.ds(off, CHUNK)], ibuf)
        cp1 = pltpu.make_async_copy(rel_hbm.at[ibuf], rb, sem)
        cp1.start()
        src = half * BATCH + base + c * CHUNK
        cp2 = pltpu.make_async_copy(d_hbm.at[pl.ds(src, CHUNK), :], db, sem)
        cp2.start()
        return (cp1, cp2)

    def compute(step):
        _, (rb, db), _ = slots[step % 2]
        half, c = divmod(step, CHUNKS_PER_HALF)
        out_ref = co_v if half == 0 else cu_v
        lane = lax.iota(jnp.int32, LANES)

        def group(g, _):
            def row_body(i, accvec):
                b = g * LANES + i
                acc = jnp.zeros((LANES,), jnp.float32)
                for j in range(SEGS):
                    sl = pl.ds(LANES * j, LANES)
                    s = db[b, sl] + rb[b, sl]
                    acc = acc + s * s
                tot = jnp.sum(acc)
                return jnp.where(lane == i, tot, accvec)

            accvec = lax.fori_loop(0, LANES, row_body,
                                   jnp.zeros((LANES,), jnp.float32))
            out_ref[pl.ds(c * CHUNK + g * LANES, LANES)] = accvec
            return 0

        lax.fori_loop(0, GROUPS, group, 0)

    cps = issue(0)
    for step in range(NSTEPS):
        nxt = issue(step + 1) if step + 1 < NSTEPS else None
        for cp in cps:
            cp.wait()
        compute(step)
        cps = nxt

    pltpu.sync_copy(co_v, correct_hbm.at[pl.ds(base, B_PER_W)])
    pltpu.sync_copy(cu_v, corrupt_hbm.at[pl.ds(base, B_PER_W)])


def _sc_mesh():
    return plsc.VectorSubcoreMesh(core_axis_name="c", subcore_axis_name="s",
                                  num_cores=NUM_CORES,
                                  num_subcores=NUM_SUBCORES)


def _sc1(idx_all, ent_pad):
    f32 = jnp.float32
    return pl.kernel(
        _sc1_body,
        out_type=jax.ShapeDtypeStruct((2 * BATCH, PAD_DIM), f32),
        mesh=_sc_mesh(),
        compiler_params=pltpu.CompilerParams(needs_layout_passes=False),
        scratch_types=(
            pltpu.VMEM((CHUNK,), jnp.int32),
            pltpu.VMEM((CHUNK,), jnp.int32),
            pltpu.VMEM((CHUNK,), jnp.int32),
            pltpu.VMEM((CHUNK,), jnp.int32),
            pltpu.VMEM((CHUNK, PAD_DIM), f32),
            pltpu.VMEM((CHUNK, PAD_DIM), f32),
            pltpu.VMEM((CHUNK, PAD_DIM), f32),
            pltpu.VMEM((CHUNK, PAD_DIM), f32),
            pltpu.SemaphoreType.DMA,
            pltpu.SemaphoreType.DMA,
        ),
    )(idx_all, ent_pad)


def _sc2(idx_all, rel_pad, d_rows):
    f32 = jnp.float32
    return pl.kernel(
        _sc2_body,
        out_type=(jax.ShapeDtypeStruct((BATCH,), f32),
                  jax.ShapeDtypeStruct((BATCH,), f32)),
        mesh=_sc_mesh(),
        compiler_params=pltpu.CompilerParams(needs_layout_passes=False),
        scratch_types=(
            pltpu.VMEM((CHUNK,), jnp.int32),
            pltpu.VMEM((CHUNK,), jnp.int32),
            pltpu.VMEM((CHUNK, PAD_DIM), f32),
            pltpu.VMEM((CHUNK, PAD_DIM), f32),
            pltpu.VMEM((CHUNK, PAD_DIM), f32),
            pltpu.VMEM((CHUNK, PAD_DIM), f32),
            pltpu.VMEM((B_PER_W,), f32),
            pltpu.VMEM((B_PER_W,), f32),
            pltpu.SemaphoreType.DMA,
            pltpu.SemaphoreType.DMA,
        ),
    )(idx_all, rel_pad, d_rows)


ROWS_BLK = 16384
NBLK = -(-100000 // ROWS_BLK)


def _pad_body(t_ref, o_ref):
    zpad = jnp.zeros((PAD_DIM - EMB_DIM, ROWS_BLK), jnp.float32)
    o_ref[...] = jnp.concatenate([t_ref[...], zpad], axis=0).T


def _pad_table(tab_t):
    n = tab_t.shape[1]
    return pl.pallas_call(
        _pad_body,
        grid=(NBLK,),
        in_specs=[pl.BlockSpec((EMB_DIM, ROWS_BLK), lambda i: (0, i))],
        out_specs=pl.BlockSpec((ROWS_BLK, PAD_DIM), lambda i: (i, 0)),
        out_shape=jax.ShapeDtypeStruct((n, PAD_DIM), jnp.float32),
    )(tab_t)


@jax.jit
def _transe(batch, corrupt_batch, ent_t, rel_t):
    b = batch.astype(jnp.int32)
    cb = corrupt_batch.astype(jnp.int32)
    idx_all = jnp.concatenate([b.T, cb.T], axis=0)
    ent_pad = _pad_table(ent_t)
    d_rows = _sc1(idx_all, ent_pad)
    rel_pad = _pad_table(rel_t)
    return _sc2(idx_all, rel_pad, d_rows)


def kernel(batch, corrupt_batch, entity_emb, relation_emb):
    return _transe(batch, corrupt_batch, entity_emb.T, relation_emb.T)


# speedup vs baseline: 1.0846x; 1.0846x over previous
import jax
import jax.numpy as jnp
from jax import lax
from jax.experimental import pallas as pl
from jax.experimental.pallas import tpu as pltpu
from jax.experimental.pallas import tpu_sc as plsc

EMB_DIM = 100
PAD_DIM = 128
BATCH = 16384

NUM_CORES = 2
NUM_SUBCORES = 16
LANES = 16
NUM_WORKERS = NUM_CORES * NUM_SUBCORES

B_PER_W = BATCH // NUM_WORKERS
CHUNK = 128
CHUNKS_PER_HALF = B_PER_W // CHUNK
GROUPS = CHUNK // LANES
SEGS = PAD_DIM // LANES


def _tec_body(idx_hbm, ent_hbm, rel_hbm, correct_hbm, corrupt_hbm,
              ih0, ir0, it0, ih1, ir1, it1,
              hb0, rb0, tb0, hb1, rb1, tb1,
              co_v, cu_v, sem0, sem1):
    wid = lax.axis_index("c") * NUM_SUBCORES + lax.axis_index("s")
    base = wid * B_PER_W

    slots = (
        ((ih0, ir0, it0), (hb0, rb0, tb0), sem0),
        ((ih1, ir1, it1), (hb1, rb1, tb1), sem1),
    )
    tables = (ent_hbm, rel_hbm, ent_hbm)

    def issue(step):
        ibufs, dbufs, sem = slots[step % 2]
        half, c = divmod(step, CHUNKS_PER_HALF)
        off = base + c * CHUNK
        cps = []
        for j in range(3):
            pltpu.sync_copy(idx_hbm.at[3 * half + j, pl.ds(off, CHUNK)],
                            ibufs[j])
            cp = pltpu.make_async_copy(tables[j].at[ibufs[j]], dbufs[j], sem)
            cp.start()
            cps.append(cp)
        return cps

    def compute(step):
        _, (hb, rb, tb), _ = slots[step % 2]
        half, c = divmod(step, CHUNKS_PER_HALF)
        out_ref = co_v if half == 0 else cu_v
        lane = lax.iota(jnp.int32, LANES)

        def group(g, _):
            def row_body(i, accvec):
                b = g * LANES + i
                acc = jnp.zeros((LANES,), jnp.float32)
                for j in range(SEGS):
                    s = (hb[b, pl.ds(LANES * j, LANES)]
                         + rb[b, pl.ds(LANES * j, LANES)]
                         - tb[b, pl.ds(LANES * j, LANES)])
                    acc = acc + s * s
                tot = jnp.sum(acc)
                return jnp.where(lane == i, tot, accvec)

            accvec = lax.fori_loop(0, LANES, row_body,
                                   jnp.zeros((LANES,), jnp.float32))
            out_ref[pl.ds(c * CHUNK + g * LANES, LANES)] = accvec
            return 0

        lax.fori_loop(0, GROUPS, group, 0)

    cps = issue(0)
    for step in range(2 * CHUNKS_PER_HALF):
        nxt = issue(step + 1) if step + 1 < 2 * CHUNKS_PER_HALF else None
        for cp in cps:
            cp.wait()
        compute(step)
        cps = nxt

    pltpu.sync_copy(co_v, correct_hbm.at[pl.ds(base, B_PER_W)])
    pltpu.sync_copy(cu_v, corrupt_hbm.at[pl.ds(base, B_PER_W)])


@jax.jit
def _transe_sc(idx_all, ent_pad, rel_pad):
    mesh = plsc.VectorSubcoreMesh(core_axis_name="c", subcore_axis_name="s",
                                  num_cores=NUM_CORES,
                                  num_subcores=NUM_SUBCORES)
    f32 = jnp.float32
    run = pl.kernel(
        _tec_body,
        out_type=(jax.ShapeDtypeStruct((BATCH,), f32),
                  jax.ShapeDtypeStruct((BATCH,), f32)),
        mesh=mesh,
        compiler_params=pltpu.CompilerParams(needs_layout_passes=False),
        scratch_types=(
            pltpu.VMEM((CHUNK,), jnp.int32),
            pltpu.VMEM((CHUNK,), jnp.int32),
            pltpu.VMEM((CHUNK,), jnp.int32),
            pltpu.VMEM((CHUNK,), jnp.int32),
            pltpu.VMEM((CHUNK,), jnp.int32),
            pltpu.VMEM((CHUNK,), jnp.int32),
            pltpu.VMEM((CHUNK, PAD_DIM), f32),
            pltpu.VMEM((CHUNK, PAD_DIM), f32),
            pltpu.VMEM((CHUNK, PAD_DIM), f32),
            pltpu.VMEM((CHUNK, PAD_DIM), f32),
            pltpu.VMEM((CHUNK, PAD_DIM), f32),
            pltpu.VMEM((CHUNK, PAD_DIM), f32),
            pltpu.VMEM((B_PER_W,), f32),
            pltpu.VMEM((B_PER_W,), f32),
            pltpu.SemaphoreType.DMA,
            pltpu.SemaphoreType.DMA,
        ),
    )
    return run(idx_all, ent_pad, rel_pad)


ROWS_BLK = 16384
NBLK = -(-100000 // ROWS_BLK)


def _pad_body(e_ref, r_ref, eo_ref, ro_ref):
    zpad = jnp.zeros((PAD_DIM - EMB_DIM, ROWS_BLK), jnp.float32)
    eo_ref[...] = jnp.concatenate([e_ref[...], zpad], axis=0).T
    ro_ref[...] = jnp.concatenate([r_ref[...], zpad], axis=0).T


@jax.jit
def _pad_tables(ent_t, rel_t):
    n = ent_t.shape[1]
    f32 = jnp.float32
    return pl.pallas_call(
        _pad_body,
        grid=(NBLK,),
        in_specs=[pl.BlockSpec((EMB_DIM, ROWS_BLK), lambda i: (0, i)),
                  pl.BlockSpec((EMB_DIM, ROWS_BLK), lambda i: (0, i))],
        out_specs=[pl.BlockSpec((ROWS_BLK, PAD_DIM), lambda i: (i, 0)),
                   pl.BlockSpec((ROWS_BLK, PAD_DIM), lambda i: (i, 0))],
        out_shape=[jax.ShapeDtypeStruct((n, PAD_DIM), f32),
                   jax.ShapeDtypeStruct((n, PAD_DIM), f32)],
    )(ent_t, rel_t)


def kernel(batch, corrupt_batch, entity_emb, relation_emb):
    b = batch.astype(jnp.int32)
    cb = corrupt_batch.astype(jnp.int32)
    idx_all = jnp.concatenate([b.T, cb.T], axis=0)
    ent_pad, rel_pad = _pad_tables(entity_emb.T, relation_emb.T)
    return _transe_sc(idx_all, ent_pad, rel_pad)
